# Initial kernel scaffold; baseline (speedup 1.0000x reference)
import functools

import jax
import jax.numpy as jnp
from jax import lax
from jax.experimental import pallas as pl
from jax.experimental.pallas import tpu as pltpu
from jax.experimental.pallas import tpu_sc as plsc

_D = 64
_NC = 2
_NS = 16
_NW = _NC * _NS
_IDXW = 128
_CHUNK = 512
_K = _CHUNK // _IDXW


def _make_lookup(n_idx):
    assert n_idx % (_NW * _CHUNK) == 0
    b_per_w = n_idx // _NW
    n_chunks = b_per_w // _CHUNK
    mesh = plsc.VectorSubcoreMesh(core_axis_name="c", subcore_axis_name="s")

    @functools.partial(
        pl.kernel,
        mesh=mesh,
        out_type=jax.ShapeDtypeStruct((n_idx, _D), jnp.float32),
        scratch_types=[
            pltpu.VMEM((_K, _IDXW), jnp.int32),
            pltpu.VMEM((_CHUNK, _D), jnp.float32),
            pltpu.SemaphoreType.DMA,
        ],
    )
    def lookup(idx_hbm, table_hbm, out_hbm, idx_v, rows_v, sem):
        wid = lax.axis_index("s") * _NC + lax.axis_index("c")
        row_base = wid * b_per_w

        def chunk_body(c, carry):
            off = row_base + c * _CHUNK
            pltpu.sync_copy(idx_hbm.at[pl.ds(off // _IDXW, _K)], idx_v)
            copies = [
                pltpu.async_copy(
                    table_hbm.at[idx_v.at[j]],
                    rows_v.at[pl.ds(j * _IDXW, _IDXW)],
                    sem,
                )
                for j in range(_K)
            ]
            for cp in copies:
                cp.wait()
            pltpu.sync_copy(rows_v, out_hbm.at[pl.ds(off, _CHUNK)])
            return carry

        lax.fori_loop(0, n_chunks, chunk_body, 0)

    return lookup


def kernel(token_ids, embeddings):
    b, l = token_ids.shape
    n = b * l
    idx2d = token_ids.reshape(n // _IDXW, _IDXW).astype(jnp.int32)
    out = _make_lookup(n)(idx2d, embeddings)
    return out.reshape(b, l, _D)


# speedup vs baseline: 1.8309x; 1.8309x over previous
import functools

import jax
import jax.numpy as jnp
from jax import lax
from jax.experimental import pallas as pl
from jax.experimental.pallas import tpu as pltpu
from jax.experimental.pallas import tpu_sc as plsc

_D = 64
_NC = 2
_NS = 16
_NW = _NC * _NS
_IDXW = 128
_CHUNK = 512
_K = _CHUNK // _IDXW


def _make_lookup(n_idx):
    assert n_idx % (_NW * _CHUNK) == 0
    b_per_w = n_idx // _NW
    rows_per_w = b_per_w // _IDXW
    n_chunks = b_per_w // _CHUNK
    mesh = plsc.VectorSubcoreMesh(core_axis_name="c", subcore_axis_name="s")

    @functools.partial(
        pl.kernel,
        mesh=mesh,
        out_type=jax.ShapeDtypeStruct((n_idx, _D), jnp.float32),
        compiler_params=pltpu.CompilerParams(use_tc_tiling_on_sc=False),
        scratch_types=[
            pltpu.VMEM((rows_per_w, _IDXW), jnp.int32),
            pltpu.VMEM((_CHUNK, _D), jnp.float32),
            pltpu.SemaphoreType.DMA,
        ],
    )
    def lookup(idx_hbm, table_hbm, out_hbm, idx_v, rows_v, sem):
        wid = lax.axis_index("s") * _NC + lax.axis_index("c")
        row_base = wid * b_per_w
        pltpu.sync_copy(idx_hbm.at[pl.ds(wid * rows_per_w, rows_per_w)], idx_v)

        def chunk_body(c, carry):
            off = row_base + c * _CHUNK
            copies = [
                pltpu.async_copy(
                    table_hbm.at[idx_v.at[c * _K + j]],
                    rows_v.at[pl.ds(j * _IDXW, _IDXW)],
                    sem,
                )
                for j in range(_K)
            ]
            for cp in copies:
                cp.wait()
            pltpu.sync_copy(rows_v, out_hbm.at[pl.ds(off, _CHUNK)])
            return carry

        lax.fori_loop(0, n_chunks, chunk_body, 0)

    return lookup


def kernel(token_ids, embeddings):
    b, l = token_ids.shape
    n = b * l
    idx2d = token_ids.reshape(n // _IDXW, _IDXW).astype(jnp.int32)
    out = _make_lookup(n)(idx2d, embeddings)
    return out.reshape(b, l, _D)


# speedup vs baseline: 1.8712x; 1.0220x over previous
import functools

import jax
import jax.numpy as jnp
from jax import lax
from jax.experimental import pallas as pl
from jax.experimental.pallas import tpu as pltpu
from jax.experimental.pallas import tpu_sc as plsc

_D = 64
_NC = 2
_NS = 16
_NW = _NC * _NS
_IDXW = 128
_CHUNK = 512
_K = _CHUNK // _IDXW


def _make_lookup(n_idx):
    assert n_idx % (_NW * _CHUNK * 2) == 0
    b_per_w = n_idx // _NW
    rows_per_w = b_per_w // _IDXW
    n_chunks = b_per_w // _CHUNK
    mesh = plsc.VectorSubcoreMesh(core_axis_name="c", subcore_axis_name="s")

    @functools.partial(
        pl.kernel,
        mesh=mesh,
        out_type=jax.ShapeDtypeStruct((n_idx, _D), jnp.float32),
        compiler_params=pltpu.CompilerParams(use_tc_tiling_on_sc=False),
        scratch_types=[
            pltpu.VMEM((rows_per_w, _IDXW), jnp.int32),
            pltpu.VMEM((_CHUNK, _D), jnp.float32),
            pltpu.VMEM((_CHUNK, _D), jnp.float32),
            pltpu.SemaphoreType.DMA,
            pltpu.SemaphoreType.DMA,
            pltpu.SemaphoreType.DMA,
            pltpu.SemaphoreType.DMA,
        ],
    )
    def lookup(idx_hbm, table_hbm, out_hbm, idx_v, rows_a, rows_b,
               gsem_a, gsem_b, wsem_a, wsem_b):
        wid = lax.axis_index("s") * _NC + lax.axis_index("c")
        row_base = wid * b_per_w
        pltpu.sync_copy(idx_hbm.at[pl.ds(wid * rows_per_w, rows_per_w)], idx_v)

        def fire_gathers(c, rows_v, sem):
            for j in range(_K):
                pltpu.async_copy(
                    table_hbm.at[idx_v.at[c * _K + j]],
                    rows_v.at[pl.ds(j * _IDXW, _IDXW)],
                    sem,
                )

        def drain_gathers(rows_v, sem):
            for j in range(_K):
                pltpu.make_async_copy(
                    table_hbm.at[idx_v.at[0]],
                    rows_v.at[pl.ds(j * _IDXW, _IDXW)],
                    sem,
                ).wait()

        def fire_write(c, rows_v, sem):
            pltpu.async_copy(
                rows_v, out_hbm.at[pl.ds(row_base + c * _CHUNK, _CHUNK)], sem)

        def drain_write(rows_v, sem):
            pltpu.make_async_copy(
                rows_v, out_hbm.at[pl.ds(row_base, _CHUNK)], sem).wait()

        fire_gathers(0, rows_a, gsem_a)

        def body(t, carry):
            ca = 2 * t
            cb = 2 * t + 1
            drain_gathers(rows_a, gsem_a)
            fire_write(ca, rows_a, wsem_a)

            @pl.when(t > 0)
            def _():
                drain_write(rows_b, wsem_b)

            fire_gathers(cb, rows_b, gsem_b)
            drain_gathers(rows_b, gsem_b)
            fire_write(cb, rows_b, wsem_b)
            drain_write(rows_a, wsem_a)
            fire_gathers(lax.rem(ca + 2, n_chunks), rows_a, gsem_a)
            return carry

        lax.fori_loop(0, n_chunks // 2, body, 0)
        drain_gathers(rows_a, gsem_a)
        drain_write(rows_b, wsem_b)

    return lookup


def kernel(token_ids, embeddings):
    b, l = token_ids.shape
    n = b * l
    idx2d = token_ids.reshape(n // _IDXW, _IDXW).astype(jnp.int32)
    out = _make_lookup(n)(idx2d, embeddings)
    return out.reshape(b, l, _D)
